# Initial kernel scaffold; baseline (speedup 1.0000x reference)
import jax
import jax.numpy as jnp
from jax.experimental import pallas as pl
from jax.experimental.pallas import tpu as pltpu

_MOVING_RATE = 0.999
_TILE = 2048


def _norm_rows(x):
    n = jnp.sqrt(jnp.sum(x * x, axis=1, keepdims=True))
    return x / jnp.maximum(n, 1e-12)


def _dot(a, b, dims):
    return jax.lax.dot_general(
        a, b, (dims, ((), ())),
        preferred_element_type=jnp.float32,
        precision=jax.lax.Precision.HIGHEST)


def _mb_kernel(key_ref, value_ref, v0_ref, mem_ref, out_ref, memout_ref):
    B, _, D = key_ref.shape
    E = mem_ref.shape[1]
    T = min(_TILE, E)
    NT = E // T

    xn = _norm_rows(key_ref[:, 0, :])
    v = value_ref[:, 0, :]

    def pass_a(t, carry):
        run_max, run_arg = carry
        m_t = mem_ref[0, pl.ds(t * T, T), :]
        mn_t = _norm_rows(m_t)
        s = _dot(xn, mn_t, ((1,), (1,)))
        tmax = jnp.max(s, axis=1)
        targ = jnp.argmax(s, axis=1).astype(jnp.int32) + t * T
        upd = tmax > run_max
        return (jnp.where(upd, tmax, run_max),
                jnp.where(upd, targ, run_arg))

    neg = jnp.full((B,), -jnp.inf, jnp.float32)
    _, embed_ind = jax.lax.fori_loop(
        0, NT, pass_a, (neg, jnp.zeros((B,), jnp.int32)))

    def pass_b(t, carry):
        run_max2, best_row = carry
        m_t = mem_ref[0, pl.ds(t * T, T), :]
        col = t * T + jax.lax.broadcasted_iota(jnp.int32, (B, T), 1)
        oneh = (embed_ind[:, None] == col).astype(jnp.float32)
        counts = jnp.sum(oneh, axis=0)
        esum = _dot(oneh, v, ((0,), (0,)))
        new_m = (m_t * _MOVING_RATE
                 + (esum / (counts[:, None] + 1e-06)) * (1.0 - _MOVING_RATE))
        memout_ref[0, pl.ds(t * T, T), :] = new_m
        mn2 = _norm_rows(new_m)
        s2 = _dot(xn, mn2, ((1,), (1,)))
        tmax2 = jnp.max(s2, axis=1)
        targl = jnp.argmax(s2, axis=1).astype(jnp.int32)
        lcol = jax.lax.broadcasted_iota(jnp.int32, (B, T), 1)
        oneh2 = (targl[:, None] == lcol).astype(jnp.float32)
        cand = _dot(oneh2, new_m, ((1,), (0,)))
        upd = tmax2 > run_max2
        return (jnp.where(upd, tmax2, run_max2),
                jnp.where(upd[:, None], cand, best_row))

    _, best_row = jax.lax.fori_loop(
        0, NT, pass_b, (neg, jnp.zeros((B, D), jnp.float32)))
    out_ref[:, 0, :] = v0_ref[...] + best_row


def kernel(key, value, memory):
    B, S, D = key.shape
    E = memory.shape[1]
    v0 = value[:, 0, :]

    out, mem = pl.pallas_call(
        _mb_kernel,
        grid=(S,),
        in_specs=[
            pl.BlockSpec((B, 1, D), lambda i: (0, i, 0)),
            pl.BlockSpec((B, 1, D), lambda i: (0, i, 0)),
            pl.BlockSpec((B, D), lambda i: (0, 0)),
            pl.BlockSpec((1, E, D), lambda i: (i, 0, 0)),
        ],
        out_specs=[
            pl.BlockSpec((B, 1, D), lambda i: (0, i, 0)),
            pl.BlockSpec((1, E, D), lambda i: (i, 0, 0)),
        ],
        out_shape=[
            jax.ShapeDtypeStruct((B, S, D), jnp.float32),
            jax.ShapeDtypeStruct((S, E, D), jnp.float32),
        ],
        compiler_params=pltpu.CompilerParams(
            dimension_semantics=("parallel",)),
    )(key, value, v0, memory)

    return (key, value, out, mem)


# speedup vs baseline: 1.1161x; 1.1161x over previous
import jax
import jax.numpy as jnp
from jax.experimental import pallas as pl
from jax.experimental.pallas import tpu as pltpu

_MOVING_RATE = 0.999
_TILE = 2048


def _norm_rows(x):
    n = jnp.sqrt(jnp.sum(x * x, axis=1, keepdims=True))
    return x / jnp.maximum(n, 1e-12)


def _dot(a, b, dims, precision=jax.lax.Precision.DEFAULT):
    return jax.lax.dot_general(
        a, b, (dims, ((), ())),
        preferred_element_type=jnp.float32,
        precision=precision)


def _mb_kernel(key_ref, value_ref, v0_ref, mem_ref, out_ref, memout_ref):
    _, B, D = key_ref.shape
    E = mem_ref.shape[1]
    T = min(_TILE, E)
    NT = E // T

    xn = _norm_rows(key_ref[0])
    v = value_ref[0]

    def pass_a(t, carry):
        run_max, run_arg = carry
        m_t = mem_ref[0, pl.ds(t * T, T), :]
        mn_t = _norm_rows(m_t)
        s = _dot(xn, mn_t, ((1,), (1,)))
        tmax = jnp.max(s, axis=1)
        targ = jnp.argmax(s, axis=1).astype(jnp.int32) + t * T
        upd = tmax > run_max
        return (jnp.where(upd, tmax, run_max),
                jnp.where(upd, targ, run_arg))

    neg = jnp.full((B,), -jnp.inf, jnp.float32)
    _, embed_ind = jax.lax.fori_loop(
        0, NT, pass_a, (neg, jnp.zeros((B,), jnp.int32)))

    def pass_b(t, carry):
        run_max2, best_row = carry
        m_t = mem_ref[0, pl.ds(t * T, T), :]
        col = t * T + jax.lax.broadcasted_iota(jnp.int32, (B, T), 1)
        oneh = (embed_ind[:, None] == col).astype(jnp.float32)
        counts = jnp.sum(oneh, axis=0)
        esum = _dot(oneh, v, ((0,), (0,)))
        new_m = (m_t * _MOVING_RATE
                 + (esum / (counts[:, None] + 1e-06)) * (1.0 - _MOVING_RATE))
        memout_ref[0, pl.ds(t * T, T), :] = new_m
        mn2 = _norm_rows(new_m)
        s2 = _dot(xn, mn2, ((1,), (1,)))
        tmax2 = jnp.max(s2, axis=1)
        targl = jnp.argmax(s2, axis=1).astype(jnp.int32)
        lcol = jax.lax.broadcasted_iota(jnp.int32, (B, T), 1)
        oneh2 = (targl[:, None] == lcol).astype(jnp.float32)
        cand = _dot(oneh2, new_m, ((1,), (0,)))
        upd = tmax2 > run_max2
        return (jnp.where(upd, tmax2, run_max2),
                jnp.where(upd[:, None], cand, best_row))

    _, best_row = jax.lax.fori_loop(
        0, NT, pass_b, (neg, jnp.zeros((B, D), jnp.float32)))
    out_ref[0] = v0_ref[...] + best_row


def kernel(key, value, memory):
    B, S, D = key.shape
    E = memory.shape[1]
    key_t = key.transpose(1, 0, 2)
    value_t = value.transpose(1, 0, 2)
    v0 = value[:, 0, :]

    out_t, mem = pl.pallas_call(
        _mb_kernel,
        grid=(S,),
        in_specs=[
            pl.BlockSpec((1, B, D), lambda i: (i, 0, 0)),
            pl.BlockSpec((1, B, D), lambda i: (i, 0, 0)),
            pl.BlockSpec((B, D), lambda i: (0, 0)),
            pl.BlockSpec((1, E, D), lambda i: (i, 0, 0)),
        ],
        out_specs=[
            pl.BlockSpec((1, B, D), lambda i: (i, 0, 0)),
            pl.BlockSpec((1, E, D), lambda i: (i, 0, 0)),
        ],
        out_shape=[
            jax.ShapeDtypeStruct((S, B, D), jnp.float32),
            jax.ShapeDtypeStruct((S, E, D), jnp.float32),
        ],
        compiler_params=pltpu.CompilerParams(
            dimension_semantics=("parallel",)),
    )(key_t, value_t, v0, memory)

    return (key, value, out_t.transpose(1, 0, 2), mem)
